# Initial kernel scaffold; baseline (speedup 1.0000x reference)
import jax
import jax.numpy as jnp
from jax.experimental import pallas as pl


def kernel(x, edge_index, W, b):
    raise NotImplementedError("write your pallas kernel here")


# speedup vs baseline: 1.3603x; 1.3603x over previous
import functools

import jax
import jax.numpy as jnp
from jax import lax
from jax.experimental import pallas as pl
from jax.experimental.pallas import tpu as pltpu
from jax.experimental.pallas import tpu_sc as plsc

N = 10000
E = 160000
D = 256

EPAD = 163840
EROWS = EPAD // 128
EC = 1024
NB = 1000

_MESH = plsc.VectorSubcoreMesh(core_axis_name="c", subcore_axis_name="s")


@functools.partial(
    pl.kernel,
    out_type=jax.ShapeDtypeStruct((EPAD, D), jnp.float32),
    mesh=_MESH,
    scratch_types=[
        pltpu.VMEM((EROWS // 32, 128), jnp.int32),
        pltpu.VMEM((8, 16), jnp.int32),
        pltpu.VMEM((128, D), jnp.float32),
        pltpu.SemaphoreType.DMA,
    ],
)
def _gather_kernel(src_hbm, y_hbm, msg_out, srcv, sidx, rows_v, sem):
    c = lax.axis_index("c")
    s = lax.axis_index("s")
    w = c * 16 + s
    rows_per_w = EROWS // 32

    pltpu.sync_copy(src_hbm.at[pl.ds(w * rows_per_w, rows_per_w)], srcv)

    def _edge_row(j, _):
        for k in range(8):
            sidx[k, pl.ds(0, 16)] = srcv[j, pl.ds(k * 16, 16)]
        cps = [
            pltpu.async_copy(y_hbm.at[sidx.at[k]],
                             rows_v.at[pl.ds(k * 16, 16)], sem)
            for k in range(8)
        ]
        for cp in cps:
            cp.wait()
        pltpu.sync_copy(rows_v,
                        msg_out.at[pl.ds((w * rows_per_w + j) * 128, 128)])
        return 0
    lax.fori_loop(0, rows_per_w, _edge_row, 0)


def _deg_body(dst_ref, o_ref):
    i = pl.program_id(0)
    j = pl.program_id(1)
    ids = lax.broadcasted_iota(jnp.int32, (NB, EC), 0) + i * NB
    oneh = (ids == dst_ref[0]).astype(jnp.float32)
    contrib = jnp.sum(oneh, axis=1, keepdims=True)

    @pl.when(j == 0)
    def _():
        o_ref[...] = contrib

    @pl.when(j > 0)
    def _():
        o_ref[...] = o_ref[...] + contrib


def _linear_body(deg_ref, x_ref, w_ref, y_ref, dis_ref):
    deg = deg_ref[:, 0] + 1.0
    dis = lax.rsqrt(deg)
    xw = jnp.dot(x_ref[...], w_ref[...], preferred_element_type=jnp.float32)
    y_ref[...] = xw * dis[:, None]
    dis_ref[...] = dis[:, None]


def _segsum_body(dst_ref, msg_ref, o_ref):
    i = pl.program_id(0)
    j = pl.program_id(1)
    ids = lax.broadcasted_iota(jnp.int32, (NB, EC), 0) + i * NB
    oneh = (ids == dst_ref[0]).astype(jnp.bfloat16)
    contrib = jnp.dot(oneh, msg_ref[...].astype(jnp.bfloat16),
                      preferred_element_type=jnp.float32)

    @pl.when(j == 0)
    def _():
        o_ref[...] = contrib

    @pl.when(j > 0)
    def _():
        o_ref[...] = o_ref[...] + contrib


def _epilogue_body(acc_ref, y_ref, dis_ref, b_ref, o_ref):
    val = dis_ref[...] * (acc_ref[...] + y_ref[...]) + b_ref[...]
    o_ref[...] = jnp.maximum(val, 0.0)


def kernel(x, edge_index, W, b):
    src = edge_index[0].astype(jnp.int32)
    dst = edge_index[1].astype(jnp.int32)
    padn = EPAD - E
    src_p = jnp.concatenate([src, jnp.zeros((padn,), jnp.int32)])
    dst_p = jnp.concatenate(
        [dst, jnp.full((padn,), N, dtype=jnp.int32)])
    src2d = src_p.reshape(EROWS, 128)
    dstch = dst_p.reshape(EPAD // EC, 1, EC)

    nblk = N // NB
    echk = EPAD // EC

    deg = pl.pallas_call(
        _deg_body,
        grid=(nblk, echk),
        in_specs=[pl.BlockSpec((1, 1, EC), lambda i, j: (j, 0, 0))],
        out_specs=pl.BlockSpec((NB, 1), lambda i, j: (i, 0)),
        out_shape=jax.ShapeDtypeStruct((N, 1), jnp.float32),
    )(dstch)

    y, dis = pl.pallas_call(
        _linear_body,
        grid=(nblk,),
        in_specs=[
            pl.BlockSpec((NB, 1), lambda i: (i, 0)),
            pl.BlockSpec((NB, D), lambda i: (i, 0)),
            pl.BlockSpec((D, D), lambda i: (0, 0)),
        ],
        out_specs=[
            pl.BlockSpec((NB, D), lambda i: (i, 0)),
            pl.BlockSpec((NB, 1), lambda i: (i, 0)),
        ],
        out_shape=[
            jax.ShapeDtypeStruct((N, D), jnp.float32),
            jax.ShapeDtypeStruct((N, 1), jnp.float32),
        ],
    )(deg, x, W)

    msg = _gather_kernel(src2d, y)

    acc = pl.pallas_call(
        _segsum_body,
        grid=(nblk, echk),
        in_specs=[
            pl.BlockSpec((1, 1, EC), lambda i, j: (j, 0, 0)),
            pl.BlockSpec((EC, D), lambda i, j: (j, 0)),
        ],
        out_specs=pl.BlockSpec((NB, D), lambda i, j: (i, 0)),
        out_shape=jax.ShapeDtypeStruct((N, D), jnp.float32),
    )(dstch, msg)

    out = pl.pallas_call(
        _epilogue_body,
        grid=(nblk,),
        in_specs=[
            pl.BlockSpec((NB, D), lambda i: (i, 0)),
            pl.BlockSpec((NB, D), lambda i: (i, 0)),
            pl.BlockSpec((NB, 1), lambda i: (i, 0)),
            pl.BlockSpec((1, D), lambda i: (0, 0)),
        ],
        out_specs=pl.BlockSpec((NB, D), lambda i: (i, 0)),
        out_shape=jax.ShapeDtypeStruct((N, D), jnp.float32),
    )(acc, y, dis, b.reshape(1, D))
    return out
